# Initial kernel scaffold; baseline (speedup 1.0000x reference)
import jax
import jax.numpy as jnp
from jax.experimental import pallas as pl


def kernel(edge_index, user_table, item_table):
    raise NotImplementedError("write your pallas kernel here")


# speedup vs baseline: 6.8657x; 6.8657x over previous
import functools

import jax
import jax.numpy as jnp
from jax import lax
from jax.experimental import pallas as pl
from jax.experimental.pallas import tpu as pltpu
from jax.experimental.pallas import tpu_sc as plsc

N_USERS = 3000
N_ITEMS = 7000
N_NODES = N_USERS + N_ITEMS
D = 128
N_LAYERS = 3
E = 320000

NC = 2
NS = 16
NW = NC * NS
EPW = E // NW
CH = 80
NCHUNK = EPW // CH

NPAD = 10240
RPS = NPAD // NS


def _sc_pass():
    mesh = plsc.VectorSubcoreMesh(core_axis_name="c", subcore_axis_name="s")

    @functools.partial(
        pl.kernel,
        mesh=mesh,
        out_type=jax.ShapeDtypeStruct((NC, NPAD, D), jnp.float32),
        scratch_types=[
            pltpu.VMEM((NCHUNK, CH), jnp.int32),
            pltpu.VMEM((NCHUNK, CH), jnp.int32),
            pltpu.VMEM((CH, D), jnp.float32),
            pltpu.VMEM_SHARED((NPAD, D), jnp.float32),
            pltpu.SemaphoreType.DMA,
        ],
    )
    def sc_pass(src_hbm, dst_hbm, x_hbm, zrow_hbm, p_out,
                src_v, dst_v, rows_v, acc_sh, sem):
        c = lax.axis_index("c")
        s = lax.axis_index("s")
        w = c * NS + s

        pltpu.sync_copy(src_hbm.at[w], src_v)
        pltpu.sync_copy(dst_hbm.at[w], dst_v)

        pltpu.sync_copy(zrow_hbm.at[pl.ds(s * RPS, RPS)],
                        acc_sh.at[pl.ds(s * RPS, RPS)])
        plsc.subcore_barrier()

        def body(j, carry):
            pltpu.async_copy(x_hbm.at[src_v.at[j]], rows_v, sem).wait()
            pltpu.sync_copy(rows_v, acc_sh.at[dst_v.at[j]], add=True)
            return carry

        lax.fori_loop(0, NCHUNK, body, 0)
        plsc.subcore_barrier()

        pltpu.sync_copy(acc_sh.at[pl.ds(s * RPS, RPS)],
                        p_out.at[c, pl.ds(s * RPS, RPS)])

    return sc_pass


def _sc_counts():
    mesh = plsc.VectorSubcoreMesh(core_axis_name="c", subcore_axis_name="s")

    @functools.partial(
        pl.kernel,
        mesh=mesh,
        out_type=jax.ShapeDtypeStruct((NC, NPAD, D), jnp.float32),
        scratch_types=[
            pltpu.VMEM((NCHUNK, CH), jnp.int32),
            pltpu.VMEM((CH, D), jnp.float32),
            pltpu.VMEM_SHARED((NPAD, D), jnp.float32),
            pltpu.SemaphoreType.DMA,
        ],
    )
    def sc_counts(dst_hbm, zcnt_hbm, ones_hbm, c_out,
                  dst_v, ones_v, cacc_sh, sem):
        c = lax.axis_index("c")
        s = lax.axis_index("s")
        w = c * NS + s

        pltpu.sync_copy(dst_hbm.at[w], dst_v)
        pltpu.sync_copy(ones_hbm, ones_v)
        pltpu.sync_copy(zcnt_hbm.at[pl.ds(s * RPS, RPS)],
                        cacc_sh.at[pl.ds(s * RPS, RPS)])
        plsc.subcore_barrier()

        def body(j, carry):
            pltpu.sync_copy(ones_v, cacc_sh.at[dst_v.at[j]], add=True)
            return carry

        lax.fori_loop(0, NCHUNK, body, 0)
        plsc.subcore_barrier()

        pltpu.sync_copy(cacc_sh.at[pl.ds(s * RPS, RPS)],
                        c_out.at[c, pl.ds(s * RPS, RPS)])

    return sc_counts


def _tc_combine(scale):
    BR = 2048

    def body(p_ref, c_ref, accin_ref, x_ref, accout_ref):
        cnt = c_ref[0, :, 0:1] + c_ref[1, :, 0:1]
        r = 1.0 / jnp.maximum(cnt, 1.0)
        x = (p_ref[0] + p_ref[1]) * r
        x_ref[...] = x
        accout_ref[...] = (accin_ref[...] + x) * scale

    return pl.pallas_call(
        body,
        grid=(NPAD // BR,),
        in_specs=[
            pl.BlockSpec((NC, BR, D), lambda i: (0, i, 0)),
            pl.BlockSpec((NC, BR, D), lambda i: (0, i, 0)),
            pl.BlockSpec((BR, D), lambda i: (i, 0)),
        ],
        out_specs=[
            pl.BlockSpec((BR, D), lambda i: (i, 0)),
            pl.BlockSpec((BR, D), lambda i: (i, 0)),
        ],
        out_shape=[
            jax.ShapeDtypeStruct((NPAD, D), jnp.float32),
            jax.ShapeDtypeStruct((NPAD, D), jnp.float32),
        ],
    )


def kernel(edge_index, user_table, item_table):
    src = edge_index[0].astype(jnp.int32).reshape(NW, NCHUNK, CH)
    dst = edge_index[1].astype(jnp.int32).reshape(NW, NCHUNK, CH)

    x0 = jnp.zeros((NPAD, D), jnp.float32)
    x0 = lax.dynamic_update_slice(x0, user_table, (0, 0))
    x0 = lax.dynamic_update_slice(x0, item_table, (N_USERS, 0))

    zrow = jnp.zeros((NPAD, D), jnp.float32)
    ones = jnp.ones((CH, D), jnp.float32)

    sc_pass = _sc_pass()
    sc_counts = _sc_counts()
    combine = _tc_combine(1.0)
    combine_last = _tc_combine(1.0 / (N_LAYERS + 1))

    cp = sc_counts(dst, zrow, ones)

    x, acc = x0, x0
    for layer in range(N_LAYERS):
        p = sc_pass(src, dst, x, zrow)
        comb = combine_last if layer == N_LAYERS - 1 else combine
        x, acc = comb(p, cp, acc)

    final = acc[:N_NODES]
    return (final[:N_USERS], final[N_USERS:])


# speedup vs baseline: 9.5832x; 1.3958x over previous
import functools

import jax
import jax.numpy as jnp
from jax import lax
from jax.experimental import pallas as pl
from jax.experimental.pallas import tpu as pltpu
from jax.experimental.pallas import tpu_sc as plsc

N_USERS = 3000
N_ITEMS = 7000
N_NODES = N_USERS + N_ITEMS
D = 128
N_LAYERS = 3
E = 320000

NC = 2
NS = 16
NW = NC * NS
EPW = E // NW
NPAD = 10240
EPWP = 10240
PADW = EPWP - EPW
CH = 128
NCHUNK = EPWP // CH
IB = 16
NBLK = NCHUNK // IB
NBUF = 4
RPS = NPAD // NS


def _sc_pass():
    mesh = plsc.VectorSubcoreMesh(core_axis_name="c", subcore_axis_name="s")

    @functools.partial(
        pl.kernel,
        mesh=mesh,
        out_type=jax.ShapeDtypeStruct((NC, NPAD, D), jnp.float32),
        scratch_types=[
            pltpu.VMEM((2, IB, CH), jnp.int32),
            pltpu.VMEM((2, IB, CH), jnp.int32),
            pltpu.VMEM((2, CH, D), jnp.float32),
            pltpu.VMEM_SHARED((NPAD, D), jnp.float32),
            pltpu.SemaphoreType.DMA,
            pltpu.SemaphoreType.DMA,
        ],
    )
    def sc_pass(src_hbm, dst_hbm, x_hbm, zrow_hbm, p_out,
                src_v, dst_v, rows_v, acc_sh, gsem, isem):
        c = lax.axis_index("c")
        s = lax.axis_index("s")
        w = c * NS + s

        pltpu.sync_copy(src_hbm.at[w, 0], src_v.at[0])
        pltpu.sync_copy(dst_hbm.at[w, 0], dst_v.at[0])
        pltpu.sync_copy(zrow_hbm.at[pl.ds(s * RPS, RPS)],
                        acc_sh.at[pl.ds(s * RPS, RPS)])
        plsc.subcore_barrier()

        def block(k, carry):
            slot = lax.rem(k, 2)
            nslot = lax.rem(k + 1, 2)

            @pl.when(k + 1 < NBLK)
            def _():
                pltpu.async_copy(src_hbm.at[w, k + 1], src_v.at[nslot], isem)
                pltpu.async_copy(dst_hbm.at[w, k + 1], dst_v.at[nslot], isem)

            pltpu.async_copy(x_hbm.at[src_v.at[slot, 0]], rows_v.at[0], gsem)
            for j in range(IB):
                cur = j % 2
                pltpu.make_async_copy(x_hbm.at[src_v.at[slot, j]],
                                      rows_v.at[cur], gsem).wait()
                if j + 1 < IB:
                    pltpu.async_copy(x_hbm.at[src_v.at[slot, j + 1]],
                                     rows_v.at[(j + 1) % 2], gsem)
                pltpu.sync_copy(rows_v.at[cur],
                                acc_sh.at[dst_v.at[slot, j]], add=True)

            @pl.when(k + 1 < NBLK)
            def _():
                pltpu.make_async_copy(src_hbm.at[w, k + 1],
                                      src_v.at[nslot], isem).wait()
                pltpu.make_async_copy(dst_hbm.at[w, k + 1],
                                      dst_v.at[nslot], isem).wait()

            return carry

        lax.fori_loop(0, NBLK, block, 0)
        plsc.subcore_barrier()

        pltpu.sync_copy(acc_sh.at[pl.ds(s * RPS, RPS)],
                        p_out.at[c, pl.ds(s * RPS, RPS)])

    return sc_pass


def _sc_counts():
    mesh = plsc.VectorSubcoreMesh(core_axis_name="c", subcore_axis_name="s")

    @functools.partial(
        pl.kernel,
        mesh=mesh,
        out_type=jax.ShapeDtypeStruct((NC, NPAD, D), jnp.float32),
        scratch_types=[
            pltpu.VMEM((NCHUNK, CH), jnp.int32),
            pltpu.VMEM((CH, D), jnp.float32),
            pltpu.VMEM_SHARED((NPAD, D), jnp.float32),
            pltpu.SemaphoreType.DMA,
        ],
    )
    def sc_counts(dst_hbm, zrow_hbm, ones_hbm, c_out,
                  dst_v, ones_v, cacc_sh, ssem):
        c = lax.axis_index("c")
        s = lax.axis_index("s")
        w = c * NS + s

        pltpu.sync_copy(dst_hbm.at[w], dst_v)
        pltpu.sync_copy(ones_hbm, ones_v)
        pltpu.sync_copy(zrow_hbm.at[pl.ds(s * RPS, RPS)],
                        cacc_sh.at[pl.ds(s * RPS, RPS)])
        plsc.subcore_barrier()

        def body(g, carry):
            base = g * NBUF
            sds = [
                pltpu.async_copy(ones_v, cacc_sh.at[dst_v.at[base + b]],
                                 ssem, add=True)
                for b in range(NBUF)
            ]
            for d in sds:
                d.wait()
            return carry

        lax.fori_loop(0, NCHUNK // NBUF, body, 0)
        plsc.subcore_barrier()

        pltpu.sync_copy(cacc_sh.at[pl.ds(s * RPS, RPS)],
                        c_out.at[c, pl.ds(s * RPS, RPS)])

    return sc_counts


def _tc_combine(scale):
    BR = 2048

    def body(p_ref, c_ref, accin_ref, x_ref, accout_ref):
        cnt = c_ref[0, :, 0:1] + c_ref[1, :, 0:1]
        r = 1.0 / jnp.maximum(cnt, 1.0)
        x = (p_ref[0] + p_ref[1]) * r
        x_ref[...] = x
        accout_ref[...] = (accin_ref[...] + x) * scale

    return pl.pallas_call(
        body,
        grid=(NPAD // BR,),
        in_specs=[
            pl.BlockSpec((NC, BR, D), lambda i: (0, i, 0)),
            pl.BlockSpec((NC, BR, D), lambda i: (0, i, 0)),
            pl.BlockSpec((BR, D), lambda i: (i, 0)),
        ],
        out_specs=[
            pl.BlockSpec((BR, D), lambda i: (i, 0)),
            pl.BlockSpec((BR, D), lambda i: (i, 0)),
        ],
        out_shape=[
            jax.ShapeDtypeStruct((NPAD, D), jnp.float32),
            jax.ShapeDtypeStruct((NPAD, D), jnp.float32),
        ],
    )


def kernel(edge_index, user_table, item_table):
    srcr = edge_index[0].astype(jnp.int32).reshape(NW, EPW)
    dstr = edge_index[1].astype(jnp.int32).reshape(NW, EPW)

    wids = jnp.arange(NW, dtype=jnp.int32)[:, None]
    pidx = jnp.arange(PADW, dtype=jnp.int32)[None, :]
    pad_src = (wids * 97 + pidx * 13) % N_NODES
    pad_dst = jnp.broadcast_to(N_NODES + pidx, (NW, PADW))

    src4 = jnp.concatenate([srcr, pad_src], axis=1).reshape(NW, NBLK, IB, CH)
    dst4 = jnp.concatenate([dstr, pad_dst], axis=1).reshape(NW, NBLK, IB, CH)
    dst3 = dst4.reshape(NW, NCHUNK, CH)

    x0 = jnp.zeros((NPAD, D), jnp.float32)
    x0 = lax.dynamic_update_slice(x0, user_table, (0, 0))
    x0 = lax.dynamic_update_slice(x0, item_table, (N_USERS, 0))

    zrow = jnp.zeros((NPAD, D), jnp.float32)
    ones = jnp.ones((CH, D), jnp.float32)

    sc_pass = _sc_pass()
    sc_counts = _sc_counts()
    combine = _tc_combine(1.0)
    combine_last = _tc_combine(1.0 / (N_LAYERS + 1))

    cp = sc_counts(dst3, zrow, ones)

    x, acc = x0, x0
    for layer in range(N_LAYERS):
        p = sc_pass(src4, dst4, x, zrow)
        comb = combine_last if layer == N_LAYERS - 1 else combine
        x, acc = comb(p, cp, acc)

    final = acc[:N_NODES]
    return (final[:N_USERS], final[N_USERS:])


# speedup vs baseline: 10.0548x; 1.0492x over previous
import functools

import jax
import jax.numpy as jnp
from jax import lax
from jax.experimental import pallas as pl
from jax.experimental.pallas import tpu as pltpu
from jax.experimental.pallas import tpu_sc as plsc

N_USERS = 3000
N_ITEMS = 7000
N_NODES = N_USERS + N_ITEMS
D = 128
N_LAYERS = 3
E = 320000

NC = 2
NS = 16
NW = NC * NS
EPW = E // NW
NPAD = 10240
EPWP = 10240
PADW = EPWP - EPW
PCH = 64
PNCHUNK = EPWP // PCH
PIB = 16
PNBLK = PNCHUNK // PIB
PNB = 4
CH = 128
NCHUNK = EPWP // CH
NBUF = 4
RPS = NPAD // NS


def _sc_pass():
    mesh = plsc.VectorSubcoreMesh(core_axis_name="c", subcore_axis_name="s")

    @functools.partial(
        pl.kernel,
        mesh=mesh,
        out_type=jax.ShapeDtypeStruct((NC, NPAD, D), jnp.float32),
        scratch_types=[
            pltpu.VMEM((2, PIB, PCH), jnp.int32),
            pltpu.VMEM((2, PIB, PCH), jnp.int32),
            pltpu.VMEM((PNB, PCH, D), jnp.float32),
            pltpu.VMEM_SHARED((NPAD, D), jnp.float32),
            pltpu.SemaphoreType.DMA,
            pltpu.SemaphoreType.DMA,
            pltpu.SemaphoreType.DMA,
            pltpu.SemaphoreType.DMA,
            pltpu.SemaphoreType.DMA,
        ],
    )
    def sc_pass(src_hbm, dst_hbm, x_hbm, zrow_hbm, p_out,
                src_v, dst_v, rows_v, acc_sh, sem0, sem1, sem2, sem3, isem):
        c = lax.axis_index("c")
        s = lax.axis_index("s")
        w = c * NS + s
        sems = [sem0, sem1, sem2, sem3]

        pltpu.sync_copy(src_hbm.at[w, 0], src_v.at[0])
        pltpu.sync_copy(dst_hbm.at[w, 0], dst_v.at[0])
        pltpu.sync_copy(zrow_hbm.at[pl.ds(s * RPS, RPS)],
                        acc_sh.at[pl.ds(s * RPS, RPS)])
        plsc.subcore_barrier()

        def gwait(slot, local, buf):
            pltpu.make_async_copy(x_hbm.at[src_v.at[slot, local]],
                                  rows_v.at[buf], sems[buf]).wait()

        def swait(slot, local, buf):
            pltpu.make_async_copy(rows_v.at[buf],
                                  acc_sh.at[dst_v.at[slot, local]],
                                  sems[buf]).wait()

        pltpu.async_copy(x_hbm.at[src_v.at[0, 0]], rows_v.at[0], sems[0])
        pltpu.async_copy(x_hbm.at[src_v.at[0, 1]], rows_v.at[1], sems[1])

        def block(k, carry):
            slot = lax.rem(k, 2)
            nslot = lax.rem(k + 1, 2)

            @pl.when(k + 1 < PNBLK)
            def _():
                pltpu.async_copy(src_hbm.at[w, k + 1], src_v.at[nslot], isem)
                pltpu.async_copy(dst_hbm.at[w, k + 1], dst_v.at[nslot], isem)

            for local in range(PIB):
                buf = local % PNB
                nbuf = (local + 2) % PNB
                gwait(slot, local, buf)
                pltpu.async_copy(rows_v.at[buf],
                                 acc_sh.at[dst_v.at[slot, local]],
                                 sems[buf], add=True)
                if local < 2:
                    @pl.when(k > 0)
                    def _(slot=slot, local=local, nbuf=nbuf):
                        swait(slot, local, nbuf)
                    pltpu.async_copy(x_hbm.at[src_v.at[slot, local + 2]],
                                     rows_v.at[nbuf], sems[nbuf])
                elif local < PIB - 2:
                    swait(slot, local, nbuf)
                    pltpu.async_copy(x_hbm.at[src_v.at[slot, local + 2]],
                                     rows_v.at[nbuf], sems[nbuf])
                else:
                    if local == PIB - 2:
                        @pl.when(k + 1 < PNBLK)
                        def _(slot=slot, nslot=nslot):
                            pltpu.make_async_copy(src_hbm.at[w, k + 1],
                                                  src_v.at[nslot], isem).wait()
                            pltpu.make_async_copy(dst_hbm.at[w, k + 1],
                                                  dst_v.at[nslot], isem).wait()
                    swait(slot, local, nbuf)

                    @pl.when(k + 1 < PNBLK)
                    def _(nslot=nslot, local=local, nbuf=nbuf):
                        pltpu.async_copy(
                            x_hbm.at[src_v.at[nslot, local + 2 - PIB]],
                            rows_v.at[nbuf], sems[nbuf])

            return carry

        lax.fori_loop(0, PNBLK, block, 0)
        swait(lax.rem(PNBLK - 1, 2), PIB - 2, (PIB - 2) % PNB)
        swait(lax.rem(PNBLK - 1, 2), PIB - 1, (PIB - 1) % PNB)
        plsc.subcore_barrier()

        pltpu.sync_copy(acc_sh.at[pl.ds(s * RPS, RPS)],
                        p_out.at[c, pl.ds(s * RPS, RPS)])

    return sc_pass


def _sc_counts():
    mesh = plsc.VectorSubcoreMesh(core_axis_name="c", subcore_axis_name="s")

    @functools.partial(
        pl.kernel,
        mesh=mesh,
        out_type=jax.ShapeDtypeStruct((NC, NPAD, D), jnp.float32),
        scratch_types=[
            pltpu.VMEM((NCHUNK, CH), jnp.int32),
            pltpu.VMEM((CH, D), jnp.float32),
            pltpu.VMEM_SHARED((NPAD, D), jnp.float32),
            pltpu.SemaphoreType.DMA,
        ],
    )
    def sc_counts(dst_hbm, zrow_hbm, ones_hbm, c_out,
                  dst_v, ones_v, cacc_sh, ssem):
        c = lax.axis_index("c")
        s = lax.axis_index("s")
        w = c * NS + s

        pltpu.sync_copy(dst_hbm.at[w], dst_v)
        pltpu.sync_copy(ones_hbm, ones_v)
        pltpu.sync_copy(zrow_hbm.at[pl.ds(s * RPS, RPS)],
                        cacc_sh.at[pl.ds(s * RPS, RPS)])
        plsc.subcore_barrier()

        def body(g, carry):
            base = g * NBUF
            sds = [
                pltpu.async_copy(ones_v, cacc_sh.at[dst_v.at[base + b]],
                                 ssem, add=True)
                for b in range(NBUF)
            ]
            for d in sds:
                d.wait()
            return carry

        lax.fori_loop(0, NCHUNK // NBUF, body, 0)
        plsc.subcore_barrier()

        pltpu.sync_copy(cacc_sh.at[pl.ds(s * RPS, RPS)],
                        c_out.at[c, pl.ds(s * RPS, RPS)])

    return sc_counts


def _tc_combine(scale):
    BR = 2048

    def body(p_ref, c_ref, accin_ref, x_ref, accout_ref):
        cnt = c_ref[0, :, 0:1] + c_ref[1, :, 0:1]
        r = 1.0 / jnp.maximum(cnt, 1.0)
        x = (p_ref[0] + p_ref[1]) * r
        x_ref[...] = x
        accout_ref[...] = (accin_ref[...] + x) * scale

    return pl.pallas_call(
        body,
        grid=(NPAD // BR,),
        in_specs=[
            pl.BlockSpec((NC, BR, D), lambda i: (0, i, 0)),
            pl.BlockSpec((NC, BR, D), lambda i: (0, i, 0)),
            pl.BlockSpec((BR, D), lambda i: (i, 0)),
        ],
        out_specs=[
            pl.BlockSpec((BR, D), lambda i: (i, 0)),
            pl.BlockSpec((BR, D), lambda i: (i, 0)),
        ],
        out_shape=[
            jax.ShapeDtypeStruct((NPAD, D), jnp.float32),
            jax.ShapeDtypeStruct((NPAD, D), jnp.float32),
        ],
    )


def kernel(edge_index, user_table, item_table):
    srcr = edge_index[0].astype(jnp.int32).reshape(NW, EPW)
    dstr = edge_index[1].astype(jnp.int32).reshape(NW, EPW)

    wids = jnp.arange(NW, dtype=jnp.int32)[:, None]
    pidx = jnp.arange(PADW, dtype=jnp.int32)[None, :]
    pad_src = (wids * 97 + pidx * 13) % N_NODES
    pad_dst = jnp.broadcast_to(N_NODES + pidx, (NW, PADW))

    src4 = jnp.concatenate([srcr, pad_src], axis=1).reshape(
        NW, PNBLK, PIB, PCH)
    dst4 = jnp.concatenate([dstr, pad_dst], axis=1).reshape(
        NW, PNBLK, PIB, PCH)
    dst3 = dst4.reshape(NW, NCHUNK, CH)

    x0 = jnp.zeros((NPAD, D), jnp.float32)
    x0 = lax.dynamic_update_slice(x0, user_table, (0, 0))
    x0 = lax.dynamic_update_slice(x0, item_table, (N_USERS, 0))

    zrow = jnp.zeros((NPAD, D), jnp.float32)
    ones = jnp.ones((CH, D), jnp.float32)

    sc_pass = _sc_pass()
    sc_counts = _sc_counts()
    combine = _tc_combine(1.0)
    combine_last = _tc_combine(1.0 / (N_LAYERS + 1))

    cp = sc_counts(dst3, zrow, ones)

    x, acc = x0, x0
    for layer in range(N_LAYERS):
        p = sc_pass(src4, dst4, x, zrow)
        comb = combine_last if layer == N_LAYERS - 1 else combine
        x, acc = comb(p, cp, acc)

    final = acc[:N_NODES]
    return (final[:N_USERS], final[N_USERS:])


# speedup vs baseline: 11.4582x; 1.1396x over previous
import functools

import jax
import jax.numpy as jnp
from jax import lax
from jax.experimental import pallas as pl
from jax.experimental.pallas import tpu as pltpu
from jax.experimental.pallas import tpu_sc as plsc

N_USERS = 3000
N_ITEMS = 7000
N_NODES = N_USERS + N_ITEMS
D = 128
N_LAYERS = 3
E = 320000

NC = 2
NS = 16
NW = NC * NS
EPW = E // NW
NPAD = 10240
EPWP = 10240
PADW = EPWP - EPW
PCH = 112
PEPW = 10080
PNCHUNK = PEPW // PCH
PIB = 6
PNBLK = PNCHUNK // PIB
PNB = 3
CH = 128
NCHUNK = EPWP // CH
NBUF = 4
RPS = NPAD // NS


def _sc_pass():
    mesh = plsc.VectorSubcoreMesh(core_axis_name="c", subcore_axis_name="s")

    @functools.partial(
        pl.kernel,
        mesh=mesh,
        out_type=jax.ShapeDtypeStruct((NC, NPAD, D), jnp.float32),
        scratch_types=[
            pltpu.VMEM((2, PIB, PCH), jnp.int32),
            pltpu.VMEM((2, PIB, PCH), jnp.int32),
            pltpu.VMEM((PNB, PCH, D), jnp.float32),
            pltpu.VMEM_SHARED((NPAD, D), jnp.float32),
            pltpu.SemaphoreType.DMA,
            pltpu.SemaphoreType.DMA,
            pltpu.SemaphoreType.DMA,
            pltpu.SemaphoreType.DMA,
        ],
    )
    def sc_pass(src_hbm, dst_hbm, x_hbm, zrow_hbm, p_out,
                src_v, dst_v, rows_v, acc_sh, sem0, sem1, sem2, isem):
        c = lax.axis_index("c")
        s = lax.axis_index("s")
        w = c * NS + s
        sems = [sem0, sem1, sem2]

        pltpu.sync_copy(src_hbm.at[w, 0], src_v.at[0])
        pltpu.sync_copy(dst_hbm.at[w, 0], dst_v.at[0])
        pltpu.sync_copy(zrow_hbm.at[pl.ds(s * RPS, RPS)],
                        acc_sh.at[pl.ds(s * RPS, RPS)])
        plsc.subcore_barrier()

        def gwait(slot, local, buf):
            pltpu.make_async_copy(x_hbm.at[src_v.at[slot, local]],
                                  rows_v.at[buf], sems[buf]).wait()

        def swait(slot, local, buf):
            pltpu.make_async_copy(rows_v.at[buf],
                                  acc_sh.at[dst_v.at[slot, local]],
                                  sems[buf]).wait()

        pltpu.async_copy(x_hbm.at[src_v.at[0, 0]], rows_v.at[0], sems[0])
        pltpu.async_copy(x_hbm.at[src_v.at[0, 1]], rows_v.at[1], sems[1])

        def block(k, carry):
            slot = lax.rem(k, 2)
            nslot = lax.rem(k + 1, 2)

            @pl.when(k + 1 < PNBLK)
            def _():
                pltpu.async_copy(src_hbm.at[w, k + 1], src_v.at[nslot], isem)
                pltpu.async_copy(dst_hbm.at[w, k + 1], dst_v.at[nslot], isem)

            for local in range(PIB):
                buf = local % PNB
                nbuf = (local + 2) % PNB
                gwait(slot, local, buf)
                pltpu.async_copy(rows_v.at[buf],
                                 acc_sh.at[dst_v.at[slot, local]],
                                 sems[buf], add=True)
                if local == 0:
                    @pl.when(k > 0)
                    def _(slot=slot, nbuf=nbuf):
                        swait(slot, 0, nbuf)
                else:
                    swait(slot, local, nbuf)
                if local < PIB - 2:
                    pltpu.async_copy(x_hbm.at[src_v.at[slot, local + 2]],
                                     rows_v.at[nbuf], sems[nbuf])
                else:
                    if local == PIB - 2:
                        @pl.when(k + 1 < PNBLK)
                        def _(nslot=nslot):
                            pltpu.make_async_copy(src_hbm.at[w, k + 1],
                                                  src_v.at[nslot], isem).wait()
                            pltpu.make_async_copy(dst_hbm.at[w, k + 1],
                                                  dst_v.at[nslot], isem).wait()

                    @pl.when(k + 1 < PNBLK)
                    def _(nslot=nslot, local=local, nbuf=nbuf):
                        pltpu.async_copy(
                            x_hbm.at[src_v.at[nslot, local + 2 - PIB]],
                            rows_v.at[nbuf], sems[nbuf])

            return carry

        lax.fori_loop(0, PNBLK, block, 0)
        swait(lax.rem(PNBLK - 1, 2), PIB - 1, (PIB - 1) % PNB)
        plsc.subcore_barrier()

        pltpu.sync_copy(acc_sh.at[pl.ds(s * RPS, RPS)],
                        p_out.at[c, pl.ds(s * RPS, RPS)])

    return sc_pass


def _sc_counts():
    mesh = plsc.VectorSubcoreMesh(core_axis_name="c", subcore_axis_name="s")

    @functools.partial(
        pl.kernel,
        mesh=mesh,
        out_type=jax.ShapeDtypeStruct((NC, NPAD, D), jnp.float32),
        scratch_types=[
            pltpu.VMEM((NCHUNK, CH), jnp.int32),
            pltpu.VMEM((CH, D), jnp.float32),
            pltpu.VMEM_SHARED((NPAD, D), jnp.float32),
            pltpu.SemaphoreType.DMA,
        ],
    )
    def sc_counts(dst_hbm, zrow_hbm, ones_hbm, c_out,
                  dst_v, ones_v, cacc_sh, ssem):
        c = lax.axis_index("c")
        s = lax.axis_index("s")
        w = c * NS + s

        pltpu.sync_copy(dst_hbm.at[w], dst_v)
        pltpu.sync_copy(ones_hbm, ones_v)
        pltpu.sync_copy(zrow_hbm.at[pl.ds(s * RPS, RPS)],
                        cacc_sh.at[pl.ds(s * RPS, RPS)])
        plsc.subcore_barrier()

        def body(g, carry):
            base = g * NBUF
            sds = [
                pltpu.async_copy(ones_v, cacc_sh.at[dst_v.at[base + b]],
                                 ssem, add=True)
                for b in range(NBUF)
            ]
            for d in sds:
                d.wait()
            return carry

        lax.fori_loop(0, NCHUNK // NBUF, body, 0)
        plsc.subcore_barrier()

        pltpu.sync_copy(cacc_sh.at[pl.ds(s * RPS, RPS)],
                        c_out.at[c, pl.ds(s * RPS, RPS)])

    return sc_counts


def _tc_combine(scale):
    BR = 2048

    def body(p_ref, c_ref, accin_ref, x_ref, accout_ref):
        cnt = c_ref[0, :, 0:1] + c_ref[1, :, 0:1]
        r = 1.0 / jnp.maximum(cnt, 1.0)
        x = (p_ref[0] + p_ref[1]) * r
        x_ref[...] = x
        accout_ref[...] = (accin_ref[...] + x) * scale

    return pl.pallas_call(
        body,
        grid=(NPAD // BR,),
        in_specs=[
            pl.BlockSpec((NC, BR, D), lambda i: (0, i, 0)),
            pl.BlockSpec((NC, BR, D), lambda i: (0, i, 0)),
            pl.BlockSpec((BR, D), lambda i: (i, 0)),
        ],
        out_specs=[
            pl.BlockSpec((BR, D), lambda i: (i, 0)),
            pl.BlockSpec((BR, D), lambda i: (i, 0)),
        ],
        out_shape=[
            jax.ShapeDtypeStruct((NPAD, D), jnp.float32),
            jax.ShapeDtypeStruct((NPAD, D), jnp.float32),
        ],
    )


def kernel(edge_index, user_table, item_table):
    srcr = edge_index[0].astype(jnp.int32).reshape(NW, EPW)
    dstr = edge_index[1].astype(jnp.int32).reshape(NW, EPW)

    wids = jnp.arange(NW, dtype=jnp.int32)[:, None]

    ppad = PEPW - EPW
    pidx = jnp.arange(ppad, dtype=jnp.int32)[None, :]
    pad_src = (wids * 97 + pidx * 13) % N_NODES
    pad_dst = jnp.broadcast_to(N_NODES + pidx, (NW, ppad))
    src4 = jnp.concatenate([srcr, pad_src], axis=1).reshape(
        NW, PNBLK, PIB, PCH)
    dst4 = jnp.concatenate([dstr, pad_dst], axis=1).reshape(
        NW, PNBLK, PIB, PCH)

    cidx = jnp.arange(PADW, dtype=jnp.int32)[None, :]
    cpad_dst = jnp.broadcast_to(N_NODES + cidx, (NW, PADW))
    dst3 = jnp.concatenate([dstr, cpad_dst], axis=1).reshape(NW, NCHUNK, CH)

    x0 = jnp.zeros((NPAD, D), jnp.float32)
    x0 = lax.dynamic_update_slice(x0, user_table, (0, 0))
    x0 = lax.dynamic_update_slice(x0, item_table, (N_USERS, 0))

    zrow = jnp.zeros((NPAD, D), jnp.float32)
    ones = jnp.ones((CH, D), jnp.float32)

    sc_pass = _sc_pass()
    sc_counts = _sc_counts()
    combine = _tc_combine(1.0)
    combine_last = _tc_combine(1.0 / (N_LAYERS + 1))

    cp = sc_counts(dst3, zrow, ones)

    x, acc = x0, x0
    for layer in range(N_LAYERS):
        p = sc_pass(src4, dst4, x, zrow)
        comb = combine_last if layer == N_LAYERS - 1 else combine
        x, acc = comb(p, cp, acc)

    final = acc[:N_NODES]
    return (final[:N_USERS], final[N_USERS:])
